# baseline (speedup 1.0000x reference)
import functools

import jax
import jax.numpy as jnp
from jax import lax
from jax.experimental import pallas as pl
from jax.experimental.pallas import tpu as pltpu
from jax.experimental.pallas import tpu_sc as plsc

B, C, N_IN = 2, 32, 786432
K_OUT = N_IN // 4

NC, NS = 2, 16

IN_CHUNK = 16384
OUT_CHUNK = IN_CHUNK // 4
N_ITERS = N_IN // IN_CHUNK
UNROLL = 8
STEPS = OUT_CHUNK // 16

_mesh = plsc.VectorSubcoreMesh(core_axis_name="c", subcore_axis_name="s")


@functools.partial(
    pl.kernel,
    out_type=jax.ShapeDtypeStruct((B, C, K_OUT), jnp.float32),
    mesh=_mesh,
    scratch_types=[
        pltpu.VMEM((IN_CHUNK,), jnp.float32),
        pltpu.VMEM((IN_CHUNK,), jnp.float32),
        pltpu.VMEM((IN_CHUNK,), jnp.float32),
        pltpu.VMEM((OUT_CHUNK,), jnp.float32),
        pltpu.VMEM((OUT_CHUNK,), jnp.float32),
        pltpu.VMEM((OUT_CHUNK,), jnp.float32),
        pltpu.SemaphoreType.DMA,
        pltpu.SemaphoreType.DMA,
        pltpu.SemaphoreType.DMA,
        pltpu.SemaphoreType.DMA,
        pltpu.SemaphoreType.DMA,
        pltpu.SemaphoreType.DMA,
    ],
    compiler_params=pltpu.CompilerParams(needs_layout_passes=False),
)
def _sc_pool4(x_hbm, out_hbm, iv0, iv1, iv2, ov0, ov1, ov2,
              si0, si1, si2, so0, so1, so2):
    w = lax.axis_index("s") * NC + lax.axis_index("c")
    in_bufs = (iv0, iv1, iv2)
    out_bufs = (ov0, ov1, ov2)
    in_sems = (si0, si1, si2)
    out_sems = (so0, so1, so2)

    iota4 = lax.iota(jnp.int32, 16) * 4

    for b in range(B):
        def in_copy(i, slot, b=b):
            return pltpu.make_async_copy(
                x_hbm.at[b, w, pl.ds(i * IN_CHUNK, IN_CHUNK)],
                in_bufs[slot],
                in_sems[slot],
            )

        def out_copy(i, slot, b=b):
            return pltpu.make_async_copy(
                out_bufs[slot],
                out_hbm.at[b, w, pl.ds(i * OUT_CHUNK, OUT_CHUNK)],
                out_sems[slot],
            )

        in_copy(0, 0).start()
        in_copy(1, 1).start()

        def one_iter(i, slot):
            in_copy(i, slot).wait()

            @pl.when(i + 2 < N_ITERS)
            def _():
                in_copy(i + 2, (slot + 2) % 3).start()

            @pl.when(i >= 3)
            def _():
                out_copy(i - 3, slot).wait()

            in_ref = in_bufs[slot]
            out_ref = out_bufs[slot]

            @plsc.parallel_loop(0, STEPS, 1, unroll=UNROLL, carry=iota4)
            def _(t, idx):
                g0 = plsc.load_gather(in_ref, [idx])
                g1 = plsc.load_gather(in_ref, [idx + 1])
                g2 = plsc.load_gather(in_ref, [idx + 2])
                g3 = plsc.load_gather(in_ref, [idx + 3])
                out_ref[pl.ds(t * 16, 16)] = jnp.maximum(
                    jnp.maximum(g0, g1), jnp.maximum(g2, g3)
                )
                return idx + 64

            out_copy(i, slot).start()

        def triple(g, carry):
            one_iter(3 * g, 0)
            one_iter(3 * g + 1, 1)
            one_iter(3 * g + 2, 2)
            return carry

        lax.fori_loop(0, N_ITERS // 3, triple, 0, unroll=False)

        out_copy(N_ITERS - 3, 0).wait()
        out_copy(N_ITERS - 2, 1).wait()
        out_copy(N_ITERS - 1, 2).wait()


def kernel(x, children_idx, cell_ids_out):
    del children_idx
    return _sc_pool4(x), cell_ids_out


# speedup vs baseline: 1.0187x; 1.0187x over previous
import functools

import jax
import jax.numpy as jnp
from jax import lax
from jax.experimental import pallas as pl
from jax.experimental.pallas import tpu as pltpu
from jax.experimental.pallas import tpu_sc as plsc

B, C, N_IN = 2, 32, 786432
K_OUT = N_IN // 4

NC, NS = 2, 16

IN_CHUNK = 32768
OUT_CHUNK = IN_CHUNK // 4
N_ITERS = N_IN // IN_CHUNK
UNROLL = 8
STEPS = OUT_CHUNK // 16

_mesh = plsc.VectorSubcoreMesh(core_axis_name="c", subcore_axis_name="s")


@functools.partial(
    pl.kernel,
    out_type=jax.ShapeDtypeStruct((B, C, K_OUT), jnp.float32),
    mesh=_mesh,
    scratch_types=[
        pltpu.VMEM((IN_CHUNK,), jnp.float32),
        pltpu.VMEM((IN_CHUNK,), jnp.float32),
        pltpu.VMEM((IN_CHUNK,), jnp.float32),
        pltpu.VMEM((OUT_CHUNK,), jnp.float32),
        pltpu.VMEM((OUT_CHUNK,), jnp.float32),
        pltpu.VMEM((OUT_CHUNK,), jnp.float32),
        pltpu.SemaphoreType.DMA,
        pltpu.SemaphoreType.DMA,
        pltpu.SemaphoreType.DMA,
        pltpu.SemaphoreType.DMA,
        pltpu.SemaphoreType.DMA,
        pltpu.SemaphoreType.DMA,
    ],
    compiler_params=pltpu.CompilerParams(needs_layout_passes=False),
)
def _sc_pool4(x_hbm, out_hbm, iv0, iv1, iv2, ov0, ov1, ov2,
              si0, si1, si2, so0, so1, so2):
    w = lax.axis_index("s") * NC + lax.axis_index("c")
    in_bufs = (iv0, iv1, iv2)
    out_bufs = (ov0, ov1, ov2)
    in_sems = (si0, si1, si2)
    out_sems = (so0, so1, so2)

    iota4 = lax.iota(jnp.int32, 16) * 4

    for b in range(B):
        def in_copy(i, slot, b=b):
            return pltpu.make_async_copy(
                x_hbm.at[b, w, pl.ds(i * IN_CHUNK, IN_CHUNK)],
                in_bufs[slot],
                in_sems[slot],
            )

        def out_copy(i, slot, b=b):
            return pltpu.make_async_copy(
                out_bufs[slot],
                out_hbm.at[b, w, pl.ds(i * OUT_CHUNK, OUT_CHUNK)],
                out_sems[slot],
            )

        in_copy(0, 0).start()
        in_copy(1, 1).start()

        def one_iter(i, slot):
            in_copy(i, slot).wait()

            @pl.when(i + 2 < N_ITERS)
            def _():
                in_copy(i + 2, (slot + 2) % 3).start()

            @pl.when(i >= 3)
            def _():
                out_copy(i - 3, slot).wait()

            in_ref = in_bufs[slot]
            out_ref = out_bufs[slot]

            @plsc.parallel_loop(0, STEPS, 1, unroll=UNROLL, carry=iota4)
            def _(t, idx):
                g0 = plsc.load_gather(in_ref, [idx])
                g1 = plsc.load_gather(in_ref, [idx + 1])
                g2 = plsc.load_gather(in_ref, [idx + 2])
                g3 = plsc.load_gather(in_ref, [idx + 3])
                out_ref[pl.ds(t * 16, 16)] = jnp.maximum(
                    jnp.maximum(g0, g1), jnp.maximum(g2, g3)
                )
                return idx + 64

            out_copy(i, slot).start()

        def triple(g, carry):
            one_iter(3 * g, 0)
            one_iter(3 * g + 1, 1)
            one_iter(3 * g + 2, 2)
            return carry

        lax.fori_loop(0, N_ITERS // 3, triple, 0, unroll=False)

        out_copy(N_ITERS - 3, 0).wait()
        out_copy(N_ITERS - 2, 1).wait()
        out_copy(N_ITERS - 1, 2).wait()


def kernel(x, children_idx, cell_ids_out):
    del children_idx
    return _sc_pool4(x), cell_ids_out
